# baseline (speedup 1.0000x reference)
import functools

import jax
import jax.numpy as jnp
from jax import lax
from jax.experimental import pallas as pl
from jax.experimental.pallas import tpu as pltpu
from jax.experimental.pallas import tpu_sc as plsc

BATCH = 16384
FIELDS = 26
EMBED_DIM = 64
B_TOTAL = BATCH * FIELDS
NC, NS = 2, 16
NW = NC * NS
B_PER_W = B_TOTAL // NW
BATCH_PER_W = BATCH // NW
WIN = 128
N_WIN = BATCH_PER_W // WIN
LANES = 16
N_STEPS = N_WIN * FIELDS

_MESH = plsc.VectorSubcoreMesh(core_axis_name="c", subcore_axis_name="s")


@functools.partial(
    pl.kernel,
    mesh=_MESH,
    compiler_params=pltpu.CompilerParams(needs_layout_passes=False),
    out_type=jax.ShapeDtypeStruct((FIELDS, EMBED_DIM, BATCH), jnp.float32),
    scratch_types=[
        pltpu.VMEM((B_PER_W,), jnp.int32),
        pltpu.VMEM((2, WIN, EMBED_DIM), jnp.float32),
        pltpu.VMEM((2, EMBED_DIM, WIN), jnp.float32),
        pltpu.SemaphoreType.DMA,
        pltpu.SemaphoreType.DMA,
        pltpu.SemaphoreType.DMA,
        pltpu.SemaphoreType.DMA,
    ],
)
def _embed_gather(idx_hbm, table_hbm, out_hbm, idx_v, rows_v, tr_v,
                  g0, g1, o0, o1):
    gsems = (g0, g1)
    osems = (o0, o1)
    wid = lax.axis_index("s") * NC + lax.axis_index("c")
    base = wid * B_PER_W
    batch_base = wid * BATCH_PER_W

    pltpu.sync_copy(idx_hbm.at[pl.ds(base, B_PER_W)], idx_v)

    lane = lax.broadcasted_iota(jnp.int32, (LANES,), 0)

    def gather_start(step, b):
        w = step // FIELDS
        f = step - w * FIELDS

        def group(g, carry):
            flat = (w * WIN + g * LANES + lane) * FIELDS + f
            vec = plsc.load_gather(idx_v, [flat])
            for l in range(LANES):
                r = vec[l]
                k = g * LANES + l
                pltpu.make_async_copy(
                    table_hbm.at[r], rows_v.at[b].at[k], gsems[b]).start()
            return carry
        lax.fori_loop(0, WIN // LANES, group, 0)

    def gather_wait(b):
        pltpu.make_async_copy(
            table_hbm.at[pl.ds(0, WIN)], rows_v.at[b], gsems[b]).wait()

    def transpose(b):
        rv = rows_v.at[b]
        tv = tr_v.at[b]
        for d in range(EMBED_DIM):
            for g in range(WIN // LANES):
                vals = plsc.load_gather(
                    rv, [g * LANES + lane, jnp.full((LANES,), d, jnp.int32)])
                tv[d, pl.ds(g * LANES, LANES)] = vals

    def out_copy(step, b):
        w = step // FIELDS
        f = step - w * FIELDS
        b0 = batch_base + w * WIN
        return pltpu.make_async_copy(
            tr_v.at[b], out_hbm.at[f].at[:, pl.ds(b0, WIN)], osems[b])

    gather_start(0, 0)
    gather_start(1, 1)

    def outer(j, carry):
        for b in range(2):
            i = j * 2 + b
            bp = 1 - b

            @pl.when(jnp.logical_and(i >= 1, i <= N_STEPS - 2))
            def _():
                gather_start(i + 1, bp)

            gather_wait(b)

            @pl.when(i >= 2)
            def _():
                out_copy(i - 2, b).wait()

            transpose(b)
            out_copy(i, b).start()
        return carry

    lax.fori_loop(0, N_STEPS // 2, outer, 0)

    for b in range(2):
        out_copy(N_STEPS - 2 + b, b).wait()


def kernel(embed_input, weight):
    idx = embed_input.reshape(-1).astype(jnp.int32)
    out_t = _embed_gather(idx, weight)
    return jnp.transpose(out_t, (2, 0, 1))


# speedup vs baseline: 1.5204x; 1.5204x over previous
import functools

import jax
import jax.numpy as jnp
from jax import lax
from jax.experimental import pallas as pl
from jax.experimental.pallas import tpu as pltpu
from jax.experimental.pallas import tpu_sc as plsc

BATCH = 16384
FIELDS = 26
EMBED_DIM = 64
B_TOTAL = BATCH * FIELDS
NC, NS = 2, 16
NW = NC * NS
B_PER_W = B_TOTAL // NW
BATCH_PER_W = BATCH // NW
NBUF = 4
CHUNK_B = 8
CHUNK = CHUNK_B * FIELDS
N_CHUNKS = BATCH_PER_W // CHUNK_B
LANES = 16

_MESH = plsc.VectorSubcoreMesh(core_axis_name="c", subcore_axis_name="s")


@functools.partial(
    pl.kernel,
    mesh=_MESH,
    out_type=jax.ShapeDtypeStruct((BATCH, FIELDS, EMBED_DIM), jnp.float32),
    scratch_types=[
        pltpu.VMEM((B_PER_W,), jnp.int32),
        pltpu.VMEM((NBUF, CHUNK, EMBED_DIM), jnp.float32),
    ]
    + [pltpu.SemaphoreType.DMA] * (2 * NBUF),
)
def _embed_gather(idx_hbm, table_hbm, out_hbm, idx_v, rows_v, *sems):
    gsems, osems = sems[:NBUF], sems[NBUF:]
    wid = lax.axis_index("s") * NC + lax.axis_index("c")
    base = wid * B_PER_W
    batch_base = wid * BATCH_PER_W

    pltpu.sync_copy(idx_hbm.at[pl.ds(base, B_PER_W)], idx_v)

    def gather_start(i, b):
        def group(g, carry):
            vec = idx_v[pl.ds(i * CHUNK + g * LANES, LANES)]
            for l in range(LANES):
                r = vec[l]
                k = g * LANES + l
                pltpu.make_async_copy(
                    table_hbm.at[r], rows_v.at[b].at[k], gsems[b]).start()
            return carry
        lax.fori_loop(0, CHUNK // LANES, group, 0)

    def gather_wait(b):
        pltpu.make_async_copy(
            table_hbm.at[pl.ds(0, CHUNK)], rows_v.at[b], gsems[b]).wait()

    def out_copy(i, b):
        b0 = batch_base + i * CHUNK_B
        return [
            pltpu.make_async_copy(
                rows_v.at[b].at[pl.ds(k * FIELDS, FIELDS)],
                out_hbm.at[b0 + k], osems[b])
            for k in range(CHUNK_B)
        ]

    def out_start(i, b):
        for c in out_copy(i, b):
            c.start()

    def out_wait(i, b):
        for c in out_copy(i, b):
            c.wait()

    for b in range(NBUF):
        gather_start(b, b)

    def outer(j, carry):
        for b in range(NBUF):
            i = j * NBUF + b
            bp = (b + NBUF - 1) % NBUF

            @pl.when(jnp.logical_and(i >= 1, i <= N_CHUNKS - NBUF))
            def _():
                out_wait(i - 1, bp)
                gather_start(i - 1 + NBUF, bp)

            gather_wait(b)
            out_start(i, b)
        return carry

    lax.fori_loop(0, N_CHUNKS // NBUF, outer, 0)

    for b in range(NBUF):
        out_wait(N_CHUNKS - NBUF + b, b)


def kernel(embed_input, weight):
    idx = embed_input.reshape(-1).astype(jnp.int32)
    return _embed_gather(idx, weight)
